# baseline (speedup 1.0000x reference)
import functools

import jax
import jax.numpy as jnp
from jax import lax
from jax.experimental import pallas as pl
from jax.experimental.pallas import tpu as pltpu
from jax.experimental.pallas import tpu_sc as plsc

B, S, D, M, V = 64, 50, 128, 2048, 100000
TOPK = 12
TV = 5000
NW = 32
ROWS_PER_W = B // NW
CHUNK = 128
LANES = 16


def _ivec_body(dh_ref, h_ref, w1_ref, b1_ref, w2_ref, b2_ref, out_ref):
    x = dh_ref[...]
    h = h_ref[0]
    hb = jnp.broadcast_to(h[:, None, :], (B, S, D))
    cat = jnp.concatenate([hb, x], axis=2).reshape(B * S, 2 * D)
    t = jnp.tanh(jnp.dot(cat, w1_ref[...],
                         preferred_element_type=jnp.float32) + b1_ref[...])
    q = (jnp.dot(t, w2_ref[...],
                 preferred_element_type=jnp.float32) + b2_ref[...])
    q = q.reshape(B, S, D)
    q = q - jnp.max(q, axis=1, keepdims=True)
    e = jnp.exp(q)
    q = e / jnp.sum(e, axis=1, keepdims=True)
    out_ref[...] = jnp.sum(q * x, axis=1)


def _scores_body(clo_ref, chi_ref, iv_ref, out_ref):
    iv = iv_ref[...]
    out_ref[:, :B] = lax.dot_general(clo_ref[...], iv, (((1,), (1,)), ((), ())),
                                     preferred_element_type=jnp.float32)
    out_ref[:, B:] = lax.dot_general(chi_ref[...], iv, (((1,), (1,)), ((), ())),
                                     preferred_element_type=jnp.float32)


def _flat_index(v, b):
    return jnp.where(v < V // 2, v * (2 * B) + b,
                     (v - V // 2) * (2 * B) + (B + b))


def _topk_body(l_ref, out_ref):
    l = l_ref[...]
    pos = lax.broadcasted_iota(jnp.int32, (B, M), 1)
    cols = []
    for _ in range(TOPK):
        v = jnp.max(l, axis=1, keepdims=True)
        idx = jnp.min(jnp.where(l == v, pos, M), axis=1, keepdims=True)
        cols.append(idx)
        l = jnp.where(pos == idx, -jnp.inf, l)
    out_ref[...] = jnp.concatenate(cols, axis=1)


def _sc_logits(scores_flat, story, gp, kb_len, ctx_len):
    mesh = plsc.VectorSubcoreMesh(core_axis_name="c", subcore_axis_name="s")

    @functools.partial(
        pl.kernel, mesh=mesh,
        out_type=jax.ShapeDtypeStruct((B, M), jnp.float32),
        scratch_types=[
            pltpu.VMEM((M,), jnp.int32),
            pltpu.VMEM((M,), jnp.int32),
            pltpu.VMEM((M,), jnp.float32),
            pltpu.VMEM((M,), jnp.float32),
            pltpu.VMEM((B, LANES), jnp.int32),
            pltpu.VMEM((B, LANES), jnp.int32),
            pltpu.SemaphoreType.DMA,
        ],
    )
    def k(scores_hbm, story_hbm, gp_hbm, kb_hbm, ctx_hbm, out_hbm,
          story_v, idx_v, sc_v, gpv, kb_v, ctx_v, sem):
        cid = lax.axis_index("c")
        sid = lax.axis_index("s")
        w = sid * 2 + cid
        pltpu.sync_copy(kb_hbm, kb_v)
        pltpu.sync_copy(ctx_hbm, ctx_v)
        for r in range(ROWS_PER_W):
            b = w * ROWS_PER_W + r
            pltpu.sync_copy(story_hbm.at[b], story_v)
            pltpu.sync_copy(gp_hbm.at[b], gpv)

            def build(j, carry):
                s16 = story_v[pl.ds(j * LANES, LANES)]
                idx_v[pl.ds(j * LANES, LANES)] = _flat_index(s16, b)
                return carry
            lax.fori_loop(0, M // LANES, build, 0)

            copies = [
                pltpu.async_copy(
                    scores_hbm.at[idx_v.at[pl.ds(j * CHUNK, CHUNK)]],
                    sc_v.at[pl.ds(j * CHUNK, CHUNK)], sem)
                for j in range(M // CHUNK)
            ]
            for cp in copies:
                cp.wait()

            kb16 = kb_v[b]
            ctx16 = ctx_v[b]


            def comp(j, carry):
                posv = j * LANES + lax.iota(jnp.int32, LANES)
                sraw = sc_v[pl.ds(j * LANES, LANES)] * gpv[pl.ds(j * LANES, LANES)]
                badm = ((posv >= kb16) & (posv < ctx16 - 1)) | (posv >= ctx16)
                xm = jnp.where(badm, jnp.float32(-1e9), sraw)
                gpv[pl.ds(j * LANES, LANES)] = 1.0 / (1.0 + jnp.exp(-xm))
                return carry
            lax.fori_loop(0, M // LANES, comp, 0)
            pltpu.sync_copy(gpv, out_hbm.at[b])

    return k(scores_flat, story, gp, kb_len, ctx_len)


def kernel(dh_outputs, dh_hidden, global_pointer, batch_size, story, domain,
           context_len, kb_len, conv_len, memory_mask, memory_story,
           W1, b1, W2, b2, C_know):
    i_vec = pl.pallas_call(
        _ivec_body,
        out_shape=jax.ShapeDtypeStruct((B, D), jnp.float32),
    )(dh_outputs, dh_hidden, W1, b1.reshape(1, D), W2, b2.reshape(1, D))

    scores2d = pl.pallas_call(
        _scores_body,
        grid=(V // (2 * TV),),
        in_specs=[pl.BlockSpec((TV, D), lambda i: (i, 0)),
                  pl.BlockSpec((TV, D), lambda i: (i + V // (2 * TV), 0)),
                  pl.BlockSpec((B, D), lambda i: (0, 0))],
        out_specs=pl.BlockSpec((TV, 2 * B), lambda i: (i, 0)),
        out_shape=jax.ShapeDtypeStruct((V // 2, 2 * B), jnp.float32),
    )(C_know, C_know, i_vec)
    scores_flat = scores2d.reshape(V * B)

    kb_b = jnp.broadcast_to(kb_len.astype(jnp.int32)[:, None], (B, LANES))
    ctx_b = jnp.broadcast_to(context_len.astype(jnp.int32)[:, None], (B, LANES))
    logits = _sc_logits(scores_flat, story, global_pointer, kb_b, ctx_b)

    toppi = pl.pallas_call(
        _topk_body,
        out_shape=jax.ShapeDtypeStruct((B, TOPK), jnp.int32),
    )(logits)
    return toppi, i_vec


# speedup vs baseline: 1.6998x; 1.6998x over previous
import functools

import jax
import jax.numpy as jnp
from jax import lax
from jax.experimental import pallas as pl
from jax.experimental.pallas import tpu as pltpu
from jax.experimental.pallas import tpu_sc as plsc

B, S, D, M, V = 64, 50, 128, 2048, 100000
TOPK = 12
NW = 32
ROWS_PER_W = B // NW
LANES = 16
KBW = 64
DC = D // LANES

_GDN = lax.GatherDimensionNumbers(
    offset_dims=(), collapsed_slice_dims=(0,), start_index_map=(0,))


def _ivec_body(dh_ref, h_ref, w1_ref, b1_ref, w2_ref, b2_ref, out_ref):
    x = dh_ref[...]
    h = h_ref[0]
    hb = jnp.broadcast_to(h[:, None, :], (B, S, D))
    cat = jnp.concatenate([hb, x], axis=2).reshape(B * S, 2 * D)
    t = jnp.tanh(jnp.dot(cat, w1_ref[...],
                         preferred_element_type=jnp.float32) + b1_ref[...])
    q = (jnp.dot(t, w2_ref[...],
                 preferred_element_type=jnp.float32) + b2_ref[...])
    q = q.reshape(B, S, D)
    q = q - jnp.max(q, axis=1, keepdims=True)
    e = jnp.exp(q)
    q = e / jnp.sum(e, axis=1, keepdims=True)
    out_ref[...] = jnp.sum(q * x, axis=1)


def _topk_body(l_ref, win_ref, ctx_ref, out_ref):
    l = l_ref[...]
    pos = lax.broadcasted_iota(jnp.int32, (B, M), 1)
    ctxm1 = ctx_ref[:, :1] - 1
    l = jnp.where(pos == ctxm1, win_ref[:, :1], l)
    cols = []
    for _ in range(TOPK):
        v = jnp.max(l, axis=1, keepdims=True)
        idx = jnp.min(jnp.where(l == v, pos, M), axis=1, keepdims=True)
        cols.append(idx)
        l = jnp.where(pos == idx, -jnp.inf, l)
    out_ref[...] = jnp.concatenate(cols, axis=1)


def _shuf(v, perm):
    return lax.gather(v, perm[:, None], _GDN, (1,),
                      mode=lax.GatherScatterMode.PROMISE_IN_BOUNDS)


def _hsum(v, lane):
    for sh in (8, 4, 2, 1):
        v = v + _shuf(v, lane ^ sh)
    return v


def _dots_16(rows_v, base, gp16, iv_chunks, lane):
    out = jnp.zeros((LANES,), jnp.float32)
    for s in range(LANES):
        acc = rows_v[base + s, pl.ds(0, LANES)] * iv_chunks[0]
        for d in range(1, DC):
            acc = acc + (rows_v[base + s, pl.ds(d * LANES, LANES)]
                         * iv_chunks[d])
        out = jnp.where(lane == s, _hsum(acc, lane), out)
    return out * gp16


def _masked_sigmoid(dots16, pos16, kb16, ctx16):
    bad = ((pos16 >= kb16) & (pos16 < ctx16 - 1)) | (pos16 >= ctx16)
    xm = jnp.where(bad, jnp.float32(-1e9), dots16)
    return 1.0 / (1.0 + jnp.exp(-xm))


def _sc_logits(c_know, story, gp, kb_len, ctx_len, i_vec):
    mesh = plsc.VectorSubcoreMesh(core_axis_name="c", subcore_axis_name="s")

    @functools.partial(
        pl.kernel, mesh=mesh,
        out_type=(jax.ShapeDtypeStruct((B, M), jnp.float32),
                  jax.ShapeDtypeStruct((B, LANES), jnp.float32)),
        scratch_types=[
            pltpu.VMEM((KBW,), jnp.int32),
            pltpu.VMEM((LANES,), jnp.int32),
            pltpu.VMEM((LANES,), jnp.int32),
            pltpu.VMEM((KBW,), jnp.float32),
            pltpu.VMEM((LANES,), jnp.float32),
            pltpu.VMEM((D,), jnp.float32),
            pltpu.VMEM((KBW, D), jnp.float32),
            pltpu.VMEM((LANES, D), jnp.float32),
            pltpu.VMEM((M,), jnp.float32),
            pltpu.VMEM((B, LANES), jnp.int32),
            pltpu.VMEM((B, LANES), jnp.int32),
            pltpu.SemaphoreType.DMA,
            pltpu.SemaphoreType.DMA,
            pltpu.SemaphoreType.DMA,
        ],
    )
    def k(c_hbm, story_hbm, gp_hbm, kb_hbm, ctx_hbm, iv_hbm,
          out_hbm, win_hbm,
          story_v, storyw_v, idxw_v, gp_v, gpw_v, ivv_v, rows_v, rowsw_v,
          outbuf, kb_v, ctx_v, sem_s, sem_w, sem_r):
        cid = lax.axis_index("c")
        sid = lax.axis_index("s")
        w = sid * 2 + cid
        pltpu.sync_copy(kb_hbm, kb_v)
        pltpu.sync_copy(ctx_hbm, ctx_v)
        lane = lax.iota(jnp.int32, LANES)
        for r in range(ROWS_PER_W):
            b = w * ROWS_PER_W + r
            kb16 = kb_v[b]
            ctx16 = ctx_v[b]

            row0 = pl.multiple_of(b * M, LANES)
            rowi = pl.multiple_of(b * D, LANES)
            cp_s = pltpu.async_copy(story_hbm.at[pl.ds(row0, KBW)], story_v, sem_s)
            cp_g = pltpu.async_copy(gp_hbm.at[pl.ds(row0, KBW)], gp_v, sem_r)
            cp_i = pltpu.async_copy(iv_hbm.at[pl.ds(rowi, D)], ivv_v, sem_r)

            idxw_v[...] = b * M + ctx16 - 1
            cp_sw = pltpu.async_copy(story_hbm.at[idxw_v], storyw_v, sem_w)
            cp_gw = pltpu.async_copy(gp_hbm.at[idxw_v], gpw_v, sem_r)

            cp_s.wait()
            cp_rows = pltpu.async_copy(c_hbm.at[story_v], rows_v, sem_r)
            cp_sw.wait()
            cp_roww = pltpu.async_copy(c_hbm.at[storyw_v], rowsw_v, sem_r)

            zero16 = jnp.zeros((LANES,), jnp.float32)

            def zero(j, carry):
                outbuf[pl.ds(j * LANES, LANES)] = zero16
                return carry
            lax.fori_loop(KBW // LANES, M // LANES, zero, 0)

            cp_g.wait()
            cp_i.wait()
            cp_gw.wait()
            cp_rows.wait()
            cp_roww.wait()

            iv_chunks = [ivv_v[pl.ds(d * LANES, LANES)] for d in range(DC)]
            for c in range(KBW // LANES):
                gp16 = gp_v[pl.ds(c * LANES, LANES)]
                dots16 = _dots_16(rows_v, c * LANES, gp16, iv_chunks, lane)
                pos16 = c * LANES + lane
                outbuf[pl.ds(c * LANES, LANES)] = _masked_sigmoid(
                    dots16, pos16, kb16, ctx16)

            gpw16 = gpw_v[pl.ds(0, LANES)]
            dotsw = _dots_16(rowsw_v, 0, gpw16, iv_chunks, lane)
            sigw = _masked_sigmoid(dotsw, ctx16 - 1, kb16, ctx16)

            pltpu.sync_copy(outbuf, out_hbm.at[b])
            gpw_v[...] = sigw
            pltpu.sync_copy(gpw_v, win_hbm.at[b])

    return k(c_know, story, gp, kb_len, ctx_len, i_vec)


def kernel(dh_outputs, dh_hidden, global_pointer, batch_size, story, domain,
           context_len, kb_len, conv_len, memory_mask, memory_story,
           W1, b1, W2, b2, C_know):
    i_vec = pl.pallas_call(
        _ivec_body,
        out_shape=jax.ShapeDtypeStruct((B, D), jnp.float32),
    )(dh_outputs, dh_hidden, W1, b1.reshape(1, D), W2, b2.reshape(1, D))

    kb_b = jnp.broadcast_to(kb_len.astype(jnp.int32)[:, None], (B, LANES))
    ctx_b = jnp.broadcast_to(context_len.astype(jnp.int32)[:, None], (B, LANES))
    logits, win = _sc_logits(C_know, story.reshape(B * M),
                             global_pointer.reshape(B * M),
                             kb_b, ctx_b, i_vec.reshape(B * D))

    toppi = pl.pallas_call(
        _topk_body,
        out_shape=jax.ShapeDtypeStruct((B, TOPK), jnp.int32),
    )(logits, win, ctx_b)
    return toppi, i_vec
